# baseline (speedup 1.0000x reference)
import functools

import jax
import jax.numpy as jnp
from jax import lax
from jax.experimental import pallas as pl
from jax.experimental.pallas import tpu as pltpu
from jax.experimental.pallas import tpu_sc as plsc

_HID = 128
_NCORES = 2
_NSUB = 16
_NW = _NCORES * _NSUB
_GROW = 128
_CHUNK_GATHERS = 2
_C = _CHUNK_GATHERS * _GROW


def _sc_embed(ids2d, table):
    n_rows = ids2d.shape[0] * _GROW
    b_per_w = n_rows // _NW
    chunks = b_per_w // _C
    pairs = chunks // 2
    mesh = plsc.VectorSubcoreMesh(core_axis_name="c", subcore_axis_name="s")

    @functools.partial(
        pl.kernel,
        mesh=mesh,
        out_type=jax.ShapeDtypeStruct((n_rows, _HID), jnp.float32),
        scratch_types=[
            pltpu.VMEM((_CHUNK_GATHERS, _GROW), jnp.int32),
            pltpu.VMEM((_CHUNK_GATHERS, _GROW), jnp.int32),
            pltpu.VMEM((_C, _HID), jnp.float32),
            pltpu.VMEM((_C, _HID), jnp.float32),
            pltpu.SemaphoreType.DMA,
            pltpu.SemaphoreType.DMA,
            pltpu.SemaphoreType.DMA,
            pltpu.SemaphoreType.DMA,
        ],
    )
    def run(ids_hbm, table_hbm, out_hbm,
            idx0, idx1, rows0, rows1, sg0, sg1, so0, so1):
        wid = lax.axis_index("s") * _NCORES + lax.axis_index("c")
        row0 = wid * b_per_w
        irow0 = wid * (b_per_w // _GROW)

        def load_ids(idx_v, chunk):
            pltpu.sync_copy(
                ids_hbm.at[pl.ds(irow0 + chunk * _CHUNK_GATHERS,
                                 _CHUNK_GATHERS)],
                idx_v)

        def fire_gathers(idx_v, rows_v, sem):
            return [
                pltpu.async_copy(
                    table_hbm.at[idx_v.at[j]],
                    rows_v.at[pl.ds(j * _GROW, _GROW)],
                    sem)
                for j in range(_CHUNK_GATHERS)
            ]

        def drain(idx_v, rows_v, sem):
            for j in range(_CHUNK_GATHERS):
                pltpu.make_async_copy(
                    table_hbm.at[idx_v.at[j]],
                    rows_v.at[pl.ds(j * _GROW, _GROW)],
                    sem).wait()

        def fire_out(rows_v, chunk, sem):
            return pltpu.async_copy(
                rows_v, out_hbm.at[pl.ds(row0 + chunk * _C, _C)], sem)

        def wait_out(rows_v, chunk, sem):
            pltpu.make_async_copy(
                rows_v, out_hbm.at[pl.ds(row0 + chunk * _C, _C)], sem).wait()

        load_ids(idx0, 0)
        fire_gathers(idx0, rows0, sg0)

        def body(p, carry):
            c0 = 2 * p
            drain(idx0, rows0, sg0)
            fire_out(rows0, c0, so0)
            @pl.when(p >= 1)
            def _():
                wait_out(rows1, c0 - 1, so1)
            load_ids(idx1, c0 + 1)
            fire_gathers(idx1, rows1, sg1)
            drain(idx1, rows1, sg1)
            fire_out(rows1, c0 + 1, so1)
            wait_out(rows0, c0, so0)

            @pl.when(p + 1 < pairs)
            def _():
                load_ids(idx0, c0 + 2)
                fire_gathers(idx0, rows0, sg0)
            return carry

        lax.fori_loop(0, pairs, body, 0)
        wait_out(rows1, chunks - 1, so1)

    return run(ids2d, table)


def kernel(input_ids, attention_mask, emb_table):
    del attention_mask
    b, l = input_ids.shape
    n = b * l
    ids2d = input_ids.astype(jnp.int32).reshape(n // _GROW, _GROW)
    out = _sc_embed(ids2d, emb_table)
    return out.reshape(b, l, _HID)


# speedup vs baseline: 3.3018x; 3.3018x over previous
import functools

import jax
import jax.numpy as jnp
from jax import lax
from jax.experimental import pallas as pl
from jax.experimental.pallas import tpu as pltpu
from jax.experimental.pallas import tpu_sc as plsc

_HID = 128
_VOCAB = 32
_NCORES = 2
_NSUB = 16
_NW = _NCORES * _NSUB
_GROW = 128
_CHUNK_GATHERS = 2
_C = _CHUNK_GATHERS * _GROW


def _sc_embed(ids2d, table):
    n_rows = ids2d.shape[0] * _GROW
    b_per_w = n_rows // _NW
    chunks = b_per_w // _C
    pairs = chunks // 2
    mesh = plsc.VectorSubcoreMesh(core_axis_name="c", subcore_axis_name="s")

    @functools.partial(
        pl.kernel,
        mesh=mesh,
        out_type=jax.ShapeDtypeStruct((n_rows, _HID), jnp.float32),
        scratch_types=[
            pltpu.VMEM((_CHUNK_GATHERS, _GROW), jnp.int32),
            pltpu.VMEM((_CHUNK_GATHERS, _GROW), jnp.int32),
            pltpu.VMEM((_C, _HID), jnp.float32),
            pltpu.VMEM((_C, _HID), jnp.float32),
            pltpu.SemaphoreType.DMA,
            pltpu.SemaphoreType.DMA,
            pltpu.SemaphoreType.DMA,
            pltpu.SemaphoreType.DMA,
        ],
    )
    def run(ids_hbm, table_hbm, out_hbm,
            idx0, idx1, rows0, rows1, sg0, sg1, so0, so1):
        wid = lax.axis_index("s") * _NCORES + lax.axis_index("c")
        row0 = wid * b_per_w
        irow0 = wid * (b_per_w // _GROW)

        tab_off = wid * _VOCAB

        def load_ids(idx_v, chunk):
            pltpu.sync_copy(
                ids_hbm.at[pl.ds(irow0 + chunk * _CHUNK_GATHERS,
                                 _CHUNK_GATHERS)],
                idx_v)
            for j in range(_CHUNK_GATHERS):
                for k in range(_GROW // 16):
                    sl = (j, pl.ds(k * 16, 16))
                    idx_v[sl] = idx_v[sl] + tab_off

        def fire_gathers(idx_v, rows_v, sem):
            return [
                pltpu.async_copy(
                    table_hbm.at[idx_v.at[j]],
                    rows_v.at[pl.ds(j * _GROW, _GROW)],
                    sem)
                for j in range(_CHUNK_GATHERS)
            ]

        def drain(idx_v, rows_v, sem):
            for j in range(_CHUNK_GATHERS):
                pltpu.make_async_copy(
                    table_hbm.at[idx_v.at[j]],
                    rows_v.at[pl.ds(j * _GROW, _GROW)],
                    sem).wait()

        def fire_out(rows_v, chunk, sem):
            return pltpu.async_copy(
                rows_v, out_hbm.at[pl.ds(row0 + chunk * _C, _C)], sem)

        def wait_out(rows_v, chunk, sem):
            pltpu.make_async_copy(
                rows_v, out_hbm.at[pl.ds(row0 + chunk * _C, _C)], sem).wait()

        load_ids(idx0, 0)
        fire_gathers(idx0, rows0, sg0)

        def body(p, carry):
            c0 = 2 * p
            drain(idx0, rows0, sg0)
            fire_out(rows0, c0, so0)
            @pl.when(p >= 1)
            def _():
                wait_out(rows1, c0 - 1, so1)
            load_ids(idx1, c0 + 1)
            fire_gathers(idx1, rows1, sg1)
            drain(idx1, rows1, sg1)
            fire_out(rows1, c0 + 1, so1)
            wait_out(rows0, c0, so0)

            @pl.when(p + 1 < pairs)
            def _():
                load_ids(idx0, c0 + 2)
                fire_gathers(idx0, rows0, sg0)
            return carry

        lax.fori_loop(0, pairs, body, 0)
        wait_out(rows1, chunks - 1, so1)

    return run(ids2d, table)


def kernel(input_ids, attention_mask, emb_table):
    del attention_mask
    b, l = input_ids.shape
    n = b * l
    ids2d = input_ids.astype(jnp.int32).reshape(n // _GROW, _GROW)
    table_rep = jnp.tile(emb_table, (_NW, 1))
    out = _sc_embed(ids2d, table_rep)
    return out.reshape(b, l, _HID)


# speedup vs baseline: 7.6186x; 2.3074x over previous
import functools

import jax
import jax.numpy as jnp
from jax import lax
from jax.experimental import pallas as pl
from jax.experimental.pallas import tpu as pltpu
from jax.experimental.pallas import tpu_sc as plsc

_HID = 128
_VOCAB = 32
_NCORES = 2
_NSUB = 16
_NW = _NCORES * _NSUB
_GROW = 128
_CHUNK_GATHERS = 2
_C = _CHUNK_GATHERS * _GROW


def _sc_embed(ids2d, table):
    n_rows = ids2d.shape[0] * _GROW
    b_per_w = n_rows // _NW
    chunks = b_per_w // _C
    pairs = chunks // 2
    mesh = plsc.VectorSubcoreMesh(core_axis_name="c", subcore_axis_name="s")

    @functools.partial(
        pl.kernel,
        mesh=mesh,
        out_type=jax.ShapeDtypeStruct((n_rows, _HID), jnp.float32),
        scratch_types=[
            pltpu.VMEM((_CHUNK_GATHERS, _GROW), jnp.int32),
            pltpu.VMEM((_CHUNK_GATHERS, _GROW), jnp.int32),
            pltpu.VMEM((_C, _HID), jnp.float32),
            pltpu.VMEM((_C, _HID), jnp.float32),
            pltpu.VMEM_SHARED((_VOCAB, _HID), jnp.float32),
            pltpu.SemaphoreType.DMA,
            pltpu.SemaphoreType.DMA,
            pltpu.SemaphoreType.DMA,
            pltpu.SemaphoreType.DMA,
        ],
    )
    def run(ids_hbm, table_hbm, out_hbm,
            idx0, idx1, rows0, rows1, table_v, sg0, sg1, so0, so1):
        wid = lax.axis_index("s") * _NCORES + lax.axis_index("c")
        row0 = wid * b_per_w
        irow0 = wid * (b_per_w // _GROW)

        @pl.when(lax.axis_index("s") == 0)
        def _():
            pltpu.sync_copy(table_hbm, table_v)

        plsc.subcore_barrier()

        def load_ids(idx_v, chunk):
            pltpu.sync_copy(
                ids_hbm.at[pl.ds(irow0 + chunk * _CHUNK_GATHERS,
                                 _CHUNK_GATHERS)],
                idx_v)

        def fire_gathers(idx_v, rows_v, sem):
            return [
                pltpu.async_copy(
                    table_v.at[idx_v.at[j]],
                    rows_v.at[pl.ds(j * _GROW, _GROW)],
                    sem)
                for j in range(_CHUNK_GATHERS)
            ]

        def drain(idx_v, rows_v, sem):
            for j in range(_CHUNK_GATHERS):
                pltpu.make_async_copy(
                    table_v.at[idx_v.at[j]],
                    rows_v.at[pl.ds(j * _GROW, _GROW)],
                    sem).wait()

        def fire_out(rows_v, chunk, sem):
            return pltpu.async_copy(
                rows_v, out_hbm.at[pl.ds(row0 + chunk * _C, _C)], sem)

        def wait_out(rows_v, chunk, sem):
            pltpu.make_async_copy(
                rows_v, out_hbm.at[pl.ds(row0 + chunk * _C, _C)], sem).wait()

        load_ids(idx0, 0)
        fire_gathers(idx0, rows0, sg0)

        def body(p, carry):
            c0 = 2 * p
            drain(idx0, rows0, sg0)
            fire_out(rows0, c0, so0)
            @pl.when(p >= 1)
            def _():
                wait_out(rows1, c0 - 1, so1)
            load_ids(idx1, c0 + 1)
            fire_gathers(idx1, rows1, sg1)
            drain(idx1, rows1, sg1)
            fire_out(rows1, c0 + 1, so1)
            wait_out(rows0, c0, so0)

            @pl.when(p + 1 < pairs)
            def _():
                load_ids(idx0, c0 + 2)
                fire_gathers(idx0, rows0, sg0)
            return carry

        lax.fori_loop(0, pairs, body, 0)
        wait_out(rows1, chunks - 1, so1)

    return run(ids2d, table)


def kernel(input_ids, attention_mask, emb_table):
    del attention_mask
    b, l = input_ids.shape
    n = b * l
    ids2d = input_ids.astype(jnp.int32).reshape(n // _GROW, _GROW)
    out = _sc_embed(ids2d, emb_table)
    return out.reshape(b, l, _HID)


# speedup vs baseline: 9.3163x; 1.2228x over previous
import functools

import jax
import jax.numpy as jnp
from jax import lax
from jax.experimental import pallas as pl
from jax.experimental.pallas import tpu as pltpu
from jax.experimental.pallas import tpu_sc as plsc

_HID = 128
_VOCAB = 32
_NCORES = 2
_NSUB = 16
_NW = _NCORES * _NSUB
_C = 128
_NBUF = 4


def _sc_embed(ids2d, table):
    n_rows = ids2d.shape[0] * _C
    b_per_w = n_rows // _NW
    chunks = b_per_w // _C
    quads = chunks // _NBUF
    mesh = plsc.VectorSubcoreMesh(core_axis_name="c", subcore_axis_name="s")

    @functools.partial(
        pl.kernel,
        mesh=mesh,
        out_type=jax.ShapeDtypeStruct((n_rows, _HID), jnp.float32),
        scratch_types=[
            pltpu.VMEM((_NBUF, _C), jnp.int32),
            pltpu.VMEM((_NBUF, _C, _HID), jnp.float32),
            pltpu.VMEM_SHARED((_VOCAB, _HID), jnp.float32),
            pltpu.SemaphoreType.DMA,
            pltpu.SemaphoreType.DMA,
            pltpu.SemaphoreType.DMA,
            pltpu.SemaphoreType.DMA,
            pltpu.SemaphoreType.DMA,
            pltpu.SemaphoreType.DMA,
            pltpu.SemaphoreType.DMA,
            pltpu.SemaphoreType.DMA,
        ],
    )
    def run(ids_hbm, table_hbm, out_hbm, idx_v, rows_v, table_s, *sems):
        sg = sems[:_NBUF]
        so = sems[_NBUF:]
        wid = lax.axis_index("s") * _NCORES + lax.axis_index("c")
        row0 = wid * b_per_w
        irow0 = row0 // _C

        @pl.when(lax.axis_index("s") == 0)
        def _():
            pltpu.sync_copy(table_hbm, table_s)

        plsc.subcore_barrier()

        def load_ids(b, chunk):
            pltpu.sync_copy(ids_hbm.at[irow0 + chunk], idx_v.at[b])

        def fire_gather(b):
            pltpu.async_copy(table_s.at[idx_v.at[b]], rows_v.at[b], sg[b])

        def wait_gather(b):
            pltpu.make_async_copy(
                table_s.at[idx_v.at[b]], rows_v.at[b], sg[b]).wait()

        def fire_out(b, chunk):
            pltpu.async_copy(
                rows_v.at[b], out_hbm.at[pl.ds((irow0 + chunk) * _C, _C)],
                so[b])

        def wait_out(b, chunk):
            pltpu.make_async_copy(
                rows_v.at[b], out_hbm.at[pl.ds((irow0 + chunk) * _C, _C)],
                so[b]).wait()

        load_ids(0, 0)
        fire_gather(0)
        load_ids(1, 1)
        fire_gather(1)

        def body(q, carry):
            c0 = q * _NBUF
            for b in range(_NBUF):
                c = c0 + b
                wait_gather(b)
                fire_out(b, c)
                bn = (b + 2) % _NBUF
                @pl.when(c >= 2)
                def _():
                    wait_out(bn, c - 2)

                @pl.when(c + 2 < chunks)
                def _():
                    load_ids(bn, c + 2)
                    fire_gather(bn)
            return carry

        lax.fori_loop(0, quads, body, 0)
        wait_out(2, chunks - 2)
        wait_out(3, chunks - 1)

    return run(ids2d, table)


def kernel(input_ids, attention_mask, emb_table):
    del attention_mask
    b, l = input_ids.shape
    n = b * l
    ids2d = input_ids.astype(jnp.int32).reshape(n // _C, _C)
    out = _sc_embed(ids2d, emb_table)
    return out.reshape(b, l, _HID)
